# Initial kernel scaffold; baseline (speedup 1.0000x reference)
import jax
import jax.numpy as jnp
from jax.experimental import pallas as pl


def kernel(p, x, o, W, b, gamma, beta):
    raise NotImplementedError("write your pallas kernel here")


# speedup vs baseline: 10.7247x; 10.7247x over previous
import functools

import jax
import jax.numpy as jnp
from jax import lax
from jax.experimental import pallas as pl
from jax.experimental.pallas import tpu as pltpu
from jax.experimental.pallas import tpu_sc as plsc

N_PTS = 16384
D_IN = 64
D_OUT = 128
NSAMPLE = 16
STRIDE = 4
M = N_PTS // STRIDE
PR = 128
PC = 128
CB = 64
N_ROWS = M * NSAMPLE
TBL_D = 128

_BIG_I32 = 1 << 30
_INF = float("inf")


def _fps_kernel(px_ref, py_ref, pz_ref,
                idx_ref, npx_ref, npy_ref, npz_ref, dists_ref):
    px = px_ref[:, :]
    py = py_ref[:, :]
    pz = pz_ref[:, :]
    rows = lax.broadcasted_iota(jnp.int32, (PR, PC), 0)
    cols = lax.broadcasted_iota(jnp.int32, (PR, PC), 1)
    fidx = rows * PC + cols
    orow = lax.broadcasted_iota(jnp.int32, (M // 128, 128), 0)
    ocol = lax.broadcasted_iota(jnp.int32, (M // 128, 128), 1)
    neg = jnp.float32(-jnp.inf)

    def extract(sel, arr):
        return jnp.max(jnp.where(sel, arr, neg))

    def write(i, idxv, vx, vy, vz):
        wm = (orow == i // 128) & (ocol == i % 128)
        idx_ref[:, :] = jnp.where(wm, idxv, idx_ref[:, :])
        npx_ref[:, :] = jnp.where(wm, vx, npx_ref[:, :])
        npy_ref[:, :] = jnp.where(wm, vy, npy_ref[:, :])
        npz_ref[:, :] = jnp.where(wm, vz, npz_ref[:, :])

    sel0 = fidx == 0
    qx = extract(sel0, px)
    qy = extract(sel0, py)
    qz = extract(sel0, pz)
    write(jnp.int32(0), jnp.int32(0), qx, qy, qz)
    dists_ref[:, :] = jnp.full((PR, PC), _INF, dtype=jnp.float32)

    def body(i, carry):
        cx, cy, cz = carry
        dx = px - cx
        dy = py - cy
        dz = pz - cz
        d = (dx * dx + dz * dz) + dy * dy
        nd = jnp.minimum(dists_ref[:, :], d)
        dists_ref[:, :] = nd
        mval = jnp.max(nd)
        nxt = jnp.min(jnp.where(nd == mval, fidx, _BIG_I32))
        sel = fidx == nxt
        nqx = extract(sel, px)
        nqy = extract(sel, py)
        nqz = extract(sel, pz)
        write(i, nxt, nqx, nqy, nqz)
        return (nqx, nqy, nqz)

    lax.fori_loop(1, M, body, (qx, qy, qz))


def _fps_call(px, py, pz):
    return pl.pallas_call(
        _fps_kernel,
        out_shape=[
            jax.ShapeDtypeStruct((M // 128, 128), jnp.int32),
            jax.ShapeDtypeStruct((M // 128, 128), jnp.float32),
            jax.ShapeDtypeStruct((M // 128, 128), jnp.float32),
            jax.ShapeDtypeStruct((M // 128, 128), jnp.float32),
        ],
        scratch_shapes=[pltpu.VMEM((PR, PC), jnp.float32)],
    )(px, py, pz)


def _topk_kernel(pxr_ref, pyr_ref, pzr_ref, ncx_ref, ncy_ref, ncz_ref,
                 knn_ref):
    px = pxr_ref[:, :]
    py = pyr_ref[:, :]
    pz = pzr_ref[:, :]
    pn2 = px * px + py * py + pz * pz
    def r(v):
        return v.astype(jnp.bfloat16).astype(jnp.float32)
    dot = (r(px) * r(ncx_ref[:, :]) + r(py) * r(ncy_ref[:, :])
           + r(pz) * r(ncz_ref[:, :]))
    dt = pn2 - 2.0 * dot
    fid = lax.broadcasted_iota(jnp.int32, (CB, N_PTS), 1)
    for it in range(NSAMPLE):
        mval = jnp.min(dt, axis=1, keepdims=True)
        cand = jnp.where(dt == mval, fid, _BIG_I32)
        sidx = jnp.min(cand, axis=1, keepdims=True)
        knn_ref[:, it:it + 1] = sidx
        dt = jnp.where(fid == sidx, _INF, dt)


def _topk_call(pxr, pyr, pzr, ncx, ncy, ncz):
    full = pl.BlockSpec((1, N_PTS), lambda i: (0, 0))
    return pl.pallas_call(
        _topk_kernel,
        grid=(M // CB,),
        in_specs=[
            full, full, full,
            pl.BlockSpec((CB, 1), lambda i: (i, 0)),
            pl.BlockSpec((CB, 1), lambda i: (i, 0)),
            pl.BlockSpec((CB, 1), lambda i: (i, 0)),
        ],
        out_specs=pl.BlockSpec((CB, NSAMPLE), lambda i: (i, 0)),
        out_shape=jax.ShapeDtypeStruct((M, NSAMPLE), jnp.int32),
    )(pxr, pyr, pzr, ncx, ncy, ncz)


_SC_NC = 2
_SC_NS = 16
_SC_NW = _SC_NC * _SC_NS
_ROWS_PER_W = N_ROWS // _SC_NW
_CHUNK = 128
_N_CHUNKS = _ROWS_PER_W // _CHUNK


def _gather_sc(flat_idx, table):
    mesh = plsc.VectorSubcoreMesh(core_axis_name="c", subcore_axis_name="s")

    @functools.partial(
        pl.kernel,
        mesh=mesh,
        out_type=jax.ShapeDtypeStruct((N_ROWS, TBL_D), jnp.float32),
        scratch_types=[
            pltpu.VMEM((_CHUNK,), jnp.int32),
            pltpu.VMEM((_CHUNK, TBL_D), jnp.float32),
            pltpu.SemaphoreType.DMA,
        ],
    )
    def gk(idx_hbm, tbl_hbm, out_hbm, idx_v, rows_v, sem):
        wid = lax.axis_index("s") * _SC_NC + lax.axis_index("c")

        def body(c, carry):
            base = wid * _ROWS_PER_W + c * _CHUNK
            pltpu.sync_copy(idx_hbm.at[pl.ds(base, _CHUNK)], idx_v)
            pltpu.async_copy(tbl_hbm.at[idx_v], rows_v, sem).wait()
            pltpu.sync_copy(rows_v, out_hbm.at[pl.ds(base, _CHUNK)])
            return carry

        lax.fori_loop(0, _N_CHUNKS, body, jnp.int32(0))

    return gk(flat_idx, table)


_PB = 128


def _pool_kernel(g_ref, ncx_ref, ncy_ref, ncz_ref, w_ref, b_ref, h_ref):
    ncx = ncx_ref[:, :][:, None, :]
    ncy = ncy_ref[:, :][:, None, :]
    ncz = ncz_ref[:, :][:, None, :]
    px = g_ref[:, :, D_IN:D_IN + 1] - ncx
    py = g_ref[:, :, D_IN + 1:D_IN + 2] - ncy
    pz = g_ref[:, :, D_IN + 2:D_IN + 3] - ncz
    nrm2 = px * px + py * py + pz * pz
    den = jnp.sqrt(jnp.max(nrm2, axis=1, keepdims=True)) + 1e-8
    ppx = jnp.max(px, axis=1, keepdims=True) / den
    ppy = jnp.max(py, axis=1, keepdims=True) / den
    ppz = jnp.max(pz, axis=1, keepdims=True) / den
    pooled_x = jnp.max(g_ref[:, :, 0:D_IN], axis=1)
    pooled = jnp.concatenate(
        [ppx[:, 0, :], ppy[:, 0, :], ppz[:, 0, :], pooled_x], axis=1)
    h = jnp.dot(pooled, w_ref[:, :], preferred_element_type=jnp.float32)
    h_ref[:, :] = h + b_ref[:, :]


def _pool_call(g, ncx, ncy, ncz, w, b):
    return pl.pallas_call(
        _pool_kernel,
        grid=(M // _PB,),
        in_specs=[
            pl.BlockSpec((_PB, NSAMPLE, TBL_D), lambda i: (i, 0, 0)),
            pl.BlockSpec((_PB, 1), lambda i: (i, 0)),
            pl.BlockSpec((_PB, 1), lambda i: (i, 0)),
            pl.BlockSpec((_PB, 1), lambda i: (i, 0)),
            pl.BlockSpec((D_IN + 3, D_OUT), lambda i: (0, 0)),
            pl.BlockSpec((1, D_OUT), lambda i: (0, 0)),
        ],
        out_specs=pl.BlockSpec((_PB, D_OUT), lambda i: (i, 0)),
        out_shape=jax.ShapeDtypeStruct((M, D_OUT), jnp.float32),
    )(g, ncx, ncy, ncz, w, b)


def _bn_kernel(h_ref, g_ref, be_ref, o_ref):
    h = h_ref[:, :]
    mu = jnp.mean(h, axis=0, keepdims=True)
    var = jnp.mean((h - mu) ** 2, axis=0, keepdims=True)
    o = (h - mu) / jnp.sqrt(var + 1e-5) * g_ref[:, :] + be_ref[:, :]
    o_ref[:, :] = jnp.maximum(o, 0.0)


def _bn_call(h, gamma, beta):
    return pl.pallas_call(
        _bn_kernel,
        out_shape=jax.ShapeDtypeStruct((M, D_OUT), jnp.float32),
    )(h, gamma, beta)


def kernel(p, x, o, W, b, gamma, beta):
    px = p[:, 0].reshape(PR, PC)
    py = p[:, 1].reshape(PR, PC)
    pz = p[:, 2].reshape(PR, PC)

    idx_pl, npx_pl, npy_pl, npz_pl = _fps_call(px, py, pz)
    ncx = npx_pl.reshape(M, 1)
    ncy = npy_pl.reshape(M, 1)
    ncz = npz_pl.reshape(M, 1)

    knn = _topk_call(p[:, 0].reshape(1, N_PTS), p[:, 1].reshape(1, N_PTS),
                     p[:, 2].reshape(1, N_PTS), ncx, ncy, ncz)
    flat_idx = knn.reshape(N_ROWS)

    table = jnp.concatenate(
        [x, p, jnp.zeros((N_PTS, TBL_D - D_IN - 3), jnp.float32)], axis=1)
    g = _gather_sc(flat_idx, table)

    h = _pool_call(g.reshape(M, NSAMPLE, TBL_D),
                   ncx, ncy, ncz, W, b.reshape(1, D_OUT))
    out = _bn_call(h, gamma.reshape(1, D_OUT), beta.reshape(1, D_OUT))

    n_p = jnp.stack([npx_pl.reshape(M), npy_pl.reshape(M),
                     npz_pl.reshape(M)], axis=1)
    n_o = jnp.array([M], dtype=jnp.int32)
    return (n_p, out, n_o)


# speedup vs baseline: 11.9050x; 1.1101x over previous
import functools

import jax
import jax.numpy as jnp
from jax import lax
from jax.experimental import pallas as pl
from jax.experimental.pallas import tpu as pltpu
from jax.experimental.pallas import tpu_sc as plsc

N_PTS = 16384
D_IN = 64
D_OUT = 128
NSAMPLE = 16
STRIDE = 4
M = N_PTS // STRIDE
PR = 128
PC = 128
CB = 64
N_ROWS = M * NSAMPLE
TBL_D = 128

_BIG_I32 = 1 << 30
_INF = float("inf")


def _fps_kernel(px_ref, py_ref, pz_ref, pxc_ref, pyc_ref, pzc_ref,
                idx_ref, npx_ref, npy_ref, npz_ref, dists_ref):
    px = px_ref[:, :]
    py = py_ref[:, :]
    pz = pz_ref[:, :]
    rows = lax.broadcasted_iota(jnp.int32, (PR, PC), 0)
    cols = lax.broadcasted_iota(jnp.int32, (PR, PC), 1)
    fidx = rows * PC + cols
    orow = lax.broadcasted_iota(jnp.int32, (M // 128, 128), 0)
    ocol = lax.broadcasted_iota(jnp.int32, (M // 128, 128), 1)

    def write(i, idxv, vx, vy, vz):
        wm = (orow == i // 128) & (ocol == i % 128)
        idx_ref[:, :] = jnp.where(wm, idxv, idx_ref[:, :])
        npx_ref[:, :] = jnp.where(wm, vx, npx_ref[:, :])
        npy_ref[:, :] = jnp.where(wm, vy, npy_ref[:, :])
        npz_ref[:, :] = jnp.where(wm, vz, npz_ref[:, :])

    qx = pxc_ref[0, 0]
    qy = pyc_ref[0, 0]
    qz = pzc_ref[0, 0]
    write(jnp.int32(0), jnp.int32(0), qx, qy, qz)
    dists_ref[:, :] = jnp.full((PR, PC), _INF, dtype=jnp.float32)

    def body(i, carry):
        cx, cy, cz = carry
        dx = px - cx
        dy = py - cy
        dz = pz - cz
        d = (dx * dx + dz * dz) + dy * dy
        nd = jnp.minimum(dists_ref[:, :], d)
        dists_ref[:, :] = nd
        mval = jnp.max(nd)
        nxt = jnp.min(jnp.where(nd == mval, fidx, _BIG_I32))
        nqx = pxc_ref[nxt, 0]
        nqy = pyc_ref[nxt, 0]
        nqz = pzc_ref[nxt, 0]
        write(i, nxt, nqx, nqy, nqz)
        return (nqx, nqy, nqz)

    lax.fori_loop(1, M, body, (qx, qy, qz))


def _fps_call(px, py, pz, pxc, pyc, pzc):
    return pl.pallas_call(
        _fps_kernel,
        out_shape=[
            jax.ShapeDtypeStruct((M // 128, 128), jnp.int32),
            jax.ShapeDtypeStruct((M // 128, 128), jnp.float32),
            jax.ShapeDtypeStruct((M // 128, 128), jnp.float32),
            jax.ShapeDtypeStruct((M // 128, 128), jnp.float32),
        ],
        scratch_shapes=[pltpu.VMEM((PR, PC), jnp.float32)],
    )(px, py, pz, pxc, pyc, pzc)


def _topk_kernel(pxr_ref, pyr_ref, pzr_ref, ncx_ref, ncy_ref, ncz_ref,
                 knn_ref):
    px = pxr_ref[:, :]
    py = pyr_ref[:, :]
    pz = pzr_ref[:, :]
    pn2 = px * px + py * py + pz * pz
    def r(v):
        return v.astype(jnp.bfloat16).astype(jnp.float32)
    dot = (r(px) * r(ncx_ref[:, :]) + r(py) * r(ncy_ref[:, :])
           + r(pz) * r(ncz_ref[:, :]))
    dt = pn2 - 2.0 * dot
    fid = lax.broadcasted_iota(jnp.int32, (CB, N_PTS), 1)
    for it in range(NSAMPLE):
        mval = jnp.min(dt, axis=1, keepdims=True)
        cand = jnp.where(dt == mval, fid, _BIG_I32)
        sidx = jnp.min(cand, axis=1, keepdims=True)
        knn_ref[:, it:it + 1] = sidx
        dt = jnp.where(fid == sidx, _INF, dt)


def _topk_call(pxr, pyr, pzr, ncx, ncy, ncz):
    full = pl.BlockSpec((1, N_PTS), lambda i: (0, 0))
    return pl.pallas_call(
        _topk_kernel,
        grid=(M // CB,),
        in_specs=[
            full, full, full,
            pl.BlockSpec((CB, 1), lambda i: (i, 0)),
            pl.BlockSpec((CB, 1), lambda i: (i, 0)),
            pl.BlockSpec((CB, 1), lambda i: (i, 0)),
        ],
        out_specs=pl.BlockSpec((CB, NSAMPLE), lambda i: (i, 0)),
        out_shape=jax.ShapeDtypeStruct((M, NSAMPLE), jnp.int32),
    )(pxr, pyr, pzr, ncx, ncy, ncz)


_SC_NC = 2
_SC_NS = 16
_SC_NW = _SC_NC * _SC_NS
_ROWS_PER_W = N_ROWS // _SC_NW
_CHUNK = 128
_N_CHUNKS = _ROWS_PER_W // _CHUNK


def _gather_sc(flat_idx, table):
    mesh = plsc.VectorSubcoreMesh(core_axis_name="c", subcore_axis_name="s")

    @functools.partial(
        pl.kernel,
        mesh=mesh,
        out_type=jax.ShapeDtypeStruct((N_ROWS, TBL_D), jnp.float32),
        scratch_types=[
            pltpu.VMEM((_CHUNK,), jnp.int32),
            pltpu.VMEM((_CHUNK, TBL_D), jnp.float32),
            pltpu.SemaphoreType.DMA,
        ],
    )
    def gk(idx_hbm, tbl_hbm, out_hbm, idx_v, rows_v, sem):
        wid = lax.axis_index("s") * _SC_NC + lax.axis_index("c")

        def body(c, carry):
            base = wid * _ROWS_PER_W + c * _CHUNK
            pltpu.sync_copy(idx_hbm.at[pl.ds(base, _CHUNK)], idx_v)
            pltpu.async_copy(tbl_hbm.at[idx_v], rows_v, sem).wait()
            pltpu.sync_copy(rows_v, out_hbm.at[pl.ds(base, _CHUNK)])
            return carry

        lax.fori_loop(0, _N_CHUNKS, body, jnp.int32(0))

    return gk(flat_idx, table)


_PB = 128


def _pool_kernel(g_ref, ncx_ref, ncy_ref, ncz_ref, w_ref, b_ref, h_ref):
    ncx = ncx_ref[:, :][:, None, :]
    ncy = ncy_ref[:, :][:, None, :]
    ncz = ncz_ref[:, :][:, None, :]
    px = g_ref[:, :, D_IN:D_IN + 1] - ncx
    py = g_ref[:, :, D_IN + 1:D_IN + 2] - ncy
    pz = g_ref[:, :, D_IN + 2:D_IN + 3] - ncz
    nrm2 = px * px + py * py + pz * pz
    den = jnp.sqrt(jnp.max(nrm2, axis=1, keepdims=True)) + 1e-8
    ppx = jnp.max(px, axis=1, keepdims=True) / den
    ppy = jnp.max(py, axis=1, keepdims=True) / den
    ppz = jnp.max(pz, axis=1, keepdims=True) / den
    pooled_x = jnp.max(g_ref[:, :, 0:D_IN], axis=1)
    pooled = jnp.concatenate(
        [ppx[:, 0, :], ppy[:, 0, :], ppz[:, 0, :], pooled_x], axis=1)
    h = jnp.dot(pooled, w_ref[:, :], preferred_element_type=jnp.float32)
    h_ref[:, :] = h + b_ref[:, :]


def _pool_call(g, ncx, ncy, ncz, w, b):
    return pl.pallas_call(
        _pool_kernel,
        grid=(M // _PB,),
        in_specs=[
            pl.BlockSpec((_PB, NSAMPLE, TBL_D), lambda i: (i, 0, 0)),
            pl.BlockSpec((_PB, 1), lambda i: (i, 0)),
            pl.BlockSpec((_PB, 1), lambda i: (i, 0)),
            pl.BlockSpec((_PB, 1), lambda i: (i, 0)),
            pl.BlockSpec((D_IN + 3, D_OUT), lambda i: (0, 0)),
            pl.BlockSpec((1, D_OUT), lambda i: (0, 0)),
        ],
        out_specs=pl.BlockSpec((_PB, D_OUT), lambda i: (i, 0)),
        out_shape=jax.ShapeDtypeStruct((M, D_OUT), jnp.float32),
    )(g, ncx, ncy, ncz, w, b)


def _bn_kernel(h_ref, g_ref, be_ref, o_ref):
    h = h_ref[:, :]
    mu = jnp.mean(h, axis=0, keepdims=True)
    var = jnp.mean((h - mu) ** 2, axis=0, keepdims=True)
    o = (h - mu) / jnp.sqrt(var + 1e-5) * g_ref[:, :] + be_ref[:, :]
    o_ref[:, :] = jnp.maximum(o, 0.0)


def _bn_call(h, gamma, beta):
    return pl.pallas_call(
        _bn_kernel,
        out_shape=jax.ShapeDtypeStruct((M, D_OUT), jnp.float32),
    )(h, gamma, beta)


def kernel(p, x, o, W, b, gamma, beta):
    px = p[:, 0].reshape(PR, PC)
    py = p[:, 1].reshape(PR, PC)
    pz = p[:, 2].reshape(PR, PC)

    idx_pl, npx_pl, npy_pl, npz_pl = _fps_call(
        px, py, pz, p[:, 0].reshape(N_PTS, 1), p[:, 1].reshape(N_PTS, 1),
        p[:, 2].reshape(N_PTS, 1))
    ncx = npx_pl.reshape(M, 1)
    ncy = npy_pl.reshape(M, 1)
    ncz = npz_pl.reshape(M, 1)

    knn = _topk_call(p[:, 0].reshape(1, N_PTS), p[:, 1].reshape(1, N_PTS),
                     p[:, 2].reshape(1, N_PTS), ncx, ncy, ncz)
    flat_idx = knn.reshape(N_ROWS)

    table = jnp.concatenate(
        [x, p, jnp.zeros((N_PTS, TBL_D - D_IN - 3), jnp.float32)], axis=1)
    g = _gather_sc(flat_idx, table)

    h = _pool_call(g.reshape(M, NSAMPLE, TBL_D),
                   ncx, ncy, ncz, W, b.reshape(1, D_OUT))
    out = _bn_call(h, gamma.reshape(1, D_OUT), beta.reshape(1, D_OUT))

    n_p = jnp.stack([npx_pl.reshape(M), npy_pl.reshape(M),
                     npz_pl.reshape(M)], axis=1)
    n_o = jnp.array([M], dtype=jnp.int32)
    return (n_p, out, n_o)
